# baseline (speedup 1.0000x reference)
import functools

import jax
import jax.numpy as jnp
from jax import lax
from jax.experimental import pallas as pl
from jax.experimental.pallas import tpu as pltpu
from jax.experimental.pallas import tpu_sc as plsc

G = 2
K = 1024
D = 256
GD = D // G
DIN = 768
B = 2
T = 512
BT = B * T

NW = 32
PPW = (G * BT) // NW
HPW = PPW // 2

TB = 128
NB = BT // TB
IR = BT // 128


def _tc_body(hs_ref, w_ref, b_ref, cv_ref, dist_ref, enc_ref, idxoff_ref,
             ppl_ref, cnt_ref):
    i = pl.program_id(0)
    h = lax.dot_general(hs_ref[...], w_ref[...], (((1,), (0,)), ((), ())),
                        preferred_element_type=jnp.float32)
    h = h + b_ref[...][None, :]
    iota_k = lax.broadcasted_iota(jnp.int32, (TB, K), 1)

    @pl.when(i == 0)
    def _():
        cnt_ref[...] = jnp.zeros((G, K), jnp.float32)

    for g in range(G):
        hg = h[:, g * GD:(g + 1) * GD]
        cg = cv_ref[g]
        cross = lax.dot_general(hg, cg, (((1,), (1,)), ((), ())),
                                precision=lax.Precision.HIGHEST,
                                preferred_element_type=jnp.float32)
        hn = jnp.sum(hg * hg, axis=1, keepdims=True)
        cn = jnp.sum(cg * cg, axis=1)[None, :]
        dist = hn - 2.0 * cross + cn
        dist_ref[g] = dist
        dmin = jnp.min(dist, axis=1, keepdims=True)
        idx = jnp.min(jnp.where(dist == dmin, iota_k, jnp.int32(2 ** 30)),
                      axis=1)
        idxoff_ref[pl.ds(g * IR + i * (TB // 128), TB // 128), :] = (
            (idx + g * K).reshape(TB // 128, 128))
        onehot = (iota_k == idx[:, None]).astype(jnp.float32)
        enc_ref[g] = onehot
        cnt_ref[g] = cnt_ref[g] + jnp.sum(onehot, axis=0)

    @pl.when(i == NB - 1)
    def _():
        ent = []
        for g in range(G):
            p = jnp.clip(cnt_ref[g] * (1.0 / BT), 1e-10, 1.0)
            ent.append(jnp.exp(-jnp.sum(p * jnp.log(p + 1e-10))))
        ppl_ref[...] = (0.5 * (ent[0] + ent[1])).reshape(1, 1)


_tc_call = pl.pallas_call(
    _tc_body,
    grid=(NB,),
    in_specs=[
        pl.BlockSpec((TB, DIN), lambda i: (i, 0)),
        pl.BlockSpec((DIN, D), lambda i: (0, 0)),
        pl.BlockSpec((D,), lambda i: (0,)),
        pl.BlockSpec((G, K, GD), lambda i: (0, 0, 0)),
    ],
    out_specs=[
        pl.BlockSpec((G, TB, K), lambda i: (0, i, 0)),
        pl.BlockSpec((G, TB, K), lambda i: (0, i, 0)),
        pl.BlockSpec((G * IR, 128), lambda i: (0, 0)),
        pl.BlockSpec((1, 1), lambda i: (0, 0)),
    ],
    scratch_shapes=[pltpu.VMEM((G, K), jnp.float32)],
    out_shape=[
        jax.ShapeDtypeStruct((G, BT, K), jnp.float32),
        jax.ShapeDtypeStruct((G, BT, K), jnp.float32),
        jax.ShapeDtypeStruct((G * IR, 128), jnp.int32),
        jax.ShapeDtypeStruct((1, 1), jnp.float32),
    ],
)


def _sc_body(idxoff_hbm, cv_hbm, quant_hbm, idx_a, idx_b, rows_a, rows_b,
             sem_a, sem_b, sem_s):
    wid = lax.axis_index("c") * 16 + lax.axis_index("s")
    base = wid * PPW
    g = wid // 16
    b = (wid % 16) // 8
    t0 = (wid % 8) * PPW
    pltpu.sync_copy(idxoff_hbm.at[pl.ds(base, HPW)], idx_a)
    pltpu.sync_copy(idxoff_hbm.at[pl.ds(base + HPW, HPW)], idx_b)
    ga = pltpu.async_copy(cv_hbm.at[idx_a], rows_a, sem_a)
    gb = pltpu.async_copy(cv_hbm.at[idx_b], rows_b, sem_b)
    ga.wait()
    sa = pltpu.async_copy(
        rows_a, quant_hbm.at[b, pl.ds(t0, HPW), pl.ds(g * GD, GD)], sem_s)
    gb.wait()
    sb = pltpu.async_copy(
        rows_b, quant_hbm.at[b, pl.ds(t0 + HPW, HPW), pl.ds(g * GD, GD)],
        sem_s)
    sa.wait()
    sb.wait()


_sc_call = functools.partial(
    pl.kernel,
    mesh=plsc.VectorSubcoreMesh(core_axis_name="c", subcore_axis_name="s"),
    out_type=jax.ShapeDtypeStruct((B, T, D), jnp.float32),
    scratch_types=[
        pltpu.VMEM((HPW,), jnp.int32),
        pltpu.VMEM((HPW,), jnp.int32),
        pltpu.VMEM((HPW, GD), jnp.float32),
        pltpu.VMEM((HPW, GD), jnp.float32),
        pltpu.SemaphoreType.DMA,
        pltpu.SemaphoreType.DMA,
        pltpu.SemaphoreType.DMA,
    ],
)(_sc_body)


def kernel(hidden_states, W, b, codevectors):
    hs2 = hidden_states.reshape(BT, DIN)
    cv2 = codevectors.reshape(G * K, GD)
    dist, enc, idxoff, ppl = _tc_call(hs2, W, b, codevectors)
    quantized = _sc_call(idxoff.reshape(G * BT), cv2)
    distances = dist.reshape(G, B, T, K)
    encodings = enc.reshape(G, B, T, K)
    return (quantized, encodings, distances, ppl[0, 0])


# speedup vs baseline: 1.0448x; 1.0448x over previous
import functools

import jax
import jax.numpy as jnp
from jax import lax
from jax.experimental import pallas as pl
from jax.experimental.pallas import tpu as pltpu
from jax.experimental.pallas import tpu_sc as plsc

G = 2
K = 1024
D = 256
GD = D // G
DIN = 768
B = 2
T = 512
BT = B * T

NW = 32
PPW = (G * BT) // NW
HPW = PPW // 2

TB = 256
NB = BT // TB
IR = BT // 128


def _tc_body(hs_ref, w_ref, b_ref, cv_ref, dist_ref, enc_ref, idxoff_ref,
             ppl_ref, cnt_ref):
    i = pl.program_id(0)
    h = lax.dot_general(hs_ref[...], w_ref[...], (((1,), (0,)), ((), ())),
                        preferred_element_type=jnp.float32)
    h = h + b_ref[...][None, :]
    iota_k = lax.broadcasted_iota(jnp.int32, (TB, K), 1)

    @pl.when(i == 0)
    def _():
        cnt_ref[...] = jnp.zeros((G, K), jnp.float32)

    for g in range(G):
        hg = h[:, g * GD:(g + 1) * GD]
        cg = cv_ref[g]
        cross = lax.dot_general(hg, cg, (((1,), (1,)), ((), ())),
                                precision=lax.Precision.HIGHEST,
                                preferred_element_type=jnp.float32)
        hn = jnp.sum(hg * hg, axis=1, keepdims=True)
        cn = jnp.sum(cg * cg, axis=1)[None, :]
        dist = hn - 2.0 * cross + cn
        dist_ref[g] = dist
        dmin = jnp.min(dist, axis=1, keepdims=True)
        idx = jnp.min(jnp.where(dist == dmin, iota_k, jnp.int32(2 ** 30)),
                      axis=1)
        idxoff_ref[pl.ds(g * IR + i * (TB // 128), TB // 128), :] = (
            (idx + g * K).reshape(TB // 128, 128))
        onehot = (iota_k == idx[:, None]).astype(jnp.float32)
        enc_ref[g] = onehot
        cnt_ref[g] = cnt_ref[g] + jnp.sum(onehot, axis=0)

    @pl.when(i == NB - 1)
    def _():
        ent = []
        for g in range(G):
            p = jnp.clip(cnt_ref[g] * (1.0 / BT), 1e-10, 1.0)
            ent.append(jnp.exp(-jnp.sum(p * jnp.log(p + 1e-10))))
        ppl_ref[...] = (0.5 * (ent[0] + ent[1])).reshape(1, 1)


_tc_call = pl.pallas_call(
    _tc_body,
    grid=(NB,),
    in_specs=[
        pl.BlockSpec((TB, DIN), lambda i: (i, 0)),
        pl.BlockSpec((DIN, D), lambda i: (0, 0)),
        pl.BlockSpec((D,), lambda i: (0,)),
        pl.BlockSpec((G, K, GD), lambda i: (0, 0, 0)),
    ],
    out_specs=[
        pl.BlockSpec((G, TB, K), lambda i: (0, i, 0)),
        pl.BlockSpec((G, TB, K), lambda i: (0, i, 0)),
        pl.BlockSpec((G * IR, 128), lambda i: (0, 0)),
        pl.BlockSpec((1, 1), lambda i: (0, 0)),
    ],
    scratch_shapes=[pltpu.VMEM((G, K), jnp.float32)],
    out_shape=[
        jax.ShapeDtypeStruct((G, BT, K), jnp.float32),
        jax.ShapeDtypeStruct((G, BT, K), jnp.float32),
        jax.ShapeDtypeStruct((G * IR, 128), jnp.int32),
        jax.ShapeDtypeStruct((1, 1), jnp.float32),
    ],
)


def _sc_body(idxoff_hbm, cv_hbm, quant_hbm, idx_a, idx_b, rows_a, rows_b,
             sem_a, sem_b, sem_s):
    wid = lax.axis_index("c") * 16 + lax.axis_index("s")
    base = wid * PPW
    g = wid // 16
    b = (wid % 16) // 8
    t0 = (wid % 8) * PPW
    pltpu.sync_copy(idxoff_hbm.at[pl.ds(base, HPW)], idx_a)
    pltpu.sync_copy(idxoff_hbm.at[pl.ds(base + HPW, HPW)], idx_b)
    ga = pltpu.async_copy(cv_hbm.at[idx_a], rows_a, sem_a)
    gb = pltpu.async_copy(cv_hbm.at[idx_b], rows_b, sem_b)
    ga.wait()
    sa = pltpu.async_copy(
        rows_a, quant_hbm.at[b, pl.ds(t0, HPW), pl.ds(g * GD, GD)], sem_s)
    gb.wait()
    sb = pltpu.async_copy(
        rows_b, quant_hbm.at[b, pl.ds(t0 + HPW, HPW), pl.ds(g * GD, GD)],
        sem_s)
    sa.wait()
    sb.wait()


_sc_call = functools.partial(
    pl.kernel,
    mesh=plsc.VectorSubcoreMesh(core_axis_name="c", subcore_axis_name="s"),
    out_type=jax.ShapeDtypeStruct((B, T, D), jnp.float32),
    scratch_types=[
        pltpu.VMEM((HPW,), jnp.int32),
        pltpu.VMEM((HPW,), jnp.int32),
        pltpu.VMEM((HPW, GD), jnp.float32),
        pltpu.VMEM((HPW, GD), jnp.float32),
        pltpu.SemaphoreType.DMA,
        pltpu.SemaphoreType.DMA,
        pltpu.SemaphoreType.DMA,
    ],
)(_sc_body)


def kernel(hidden_states, W, b, codevectors):
    hs2 = hidden_states.reshape(BT, DIN)
    cv2 = codevectors.reshape(G * K, GD)
    dist, enc, idxoff, ppl = _tc_call(hs2, W, b, codevectors)
    quantized = _sc_call(idxoff.reshape(G * BT), cv2)
    distances = dist.reshape(G, B, T, K)
    encodings = enc.reshape(G, B, T, K)
    return (quantized, encodings, distances, ppl[0, 0])


# speedup vs baseline: 1.0564x; 1.0112x over previous
import functools

import jax
import jax.numpy as jnp
from jax import lax
from jax.experimental import pallas as pl
from jax.experimental.pallas import tpu as pltpu
from jax.experimental.pallas import tpu_sc as plsc

G = 2
K = 1024
D = 256
GD = D // G
DIN = 768
B = 2
T = 512
BT = B * T

NW = 32
PPW = (G * BT) // NW
HPW = PPW // 2

TB = 256
NB = BT // TB
IR = BT // 128


def _tc_body(hs_ref, w_ref, b_ref, cv_ref, dist_ref, enc_ref, idxoff_ref,
             ppl_ref, cnt_ref):
    i = pl.program_id(0)
    h = lax.dot_general(hs_ref[...], w_ref[...], (((1,), (0,)), ((), ())),
                        preferred_element_type=jnp.float32)
    h = h + b_ref[...][None, :]
    iota_k = lax.broadcasted_iota(jnp.int32, (TB, K), 1)

    @pl.when(i == 0)
    def _():
        cnt_ref[...] = jnp.zeros((G, K), jnp.float32)

    for g in range(G):
        hg = h[:, g * GD:(g + 1) * GD]
        cg = cv_ref[g]
        cross = lax.dot_general(hg, cg, (((1,), (1,)), ((), ())),
                                precision=lax.Precision.HIGHEST,
                                preferred_element_type=jnp.float32)
        hn = jnp.sum(hg * hg, axis=1, keepdims=True)
        cn = jnp.sum(cg * cg, axis=1)[None, :]
        dist = hn - 2.0 * cross + cn
        dist_ref[g] = dist
        dmin = jnp.min(dist, axis=1, keepdims=True)
        idx = jnp.min(jnp.where(dist == dmin, iota_k, jnp.int32(2 ** 30)),
                      axis=1)
        idxoff_ref[pl.ds(g * IR + i * (TB // 128), TB // 128), :] = (
            (idx + g * K).reshape(TB // 128, 128))
        onehot = (iota_k == idx[:, None]).astype(jnp.float32)
        enc_ref[g] = onehot
        cnt_ref[g] = cnt_ref[g] + jnp.sum(onehot, axis=0)

    @pl.when(i == NB - 1)
    def _():
        ent = []
        for g in range(G):
            p = jnp.clip(cnt_ref[g] * (1.0 / BT), 1e-10, 1.0)
            ent.append(jnp.exp(-jnp.sum(p * jnp.log(p + 1e-10))))
        ppl_ref[...] = (0.5 * (ent[0] + ent[1])).reshape(1, 1)


_tc_call = pl.pallas_call(
    _tc_body,
    grid=(NB,),
    in_specs=[
        pl.BlockSpec((TB, DIN), lambda i: (i, 0)),
        pl.BlockSpec((DIN, D), lambda i: (0, 0)),
        pl.BlockSpec((D,), lambda i: (0,)),
        pl.BlockSpec((G, K, GD), lambda i: (0, 0, 0)),
    ],
    out_specs=[
        pl.BlockSpec((G, TB, K), lambda i: (0, i, 0)),
        pl.BlockSpec((G, TB, K), lambda i: (0, i, 0)),
        pl.BlockSpec((G * IR, 128), lambda i: (0, 0)),
        pl.BlockSpec((1, 1), lambda i: (0, 0)),
    ],
    scratch_shapes=[pltpu.VMEM((G, K), jnp.float32)],
    out_shape=[
        jax.ShapeDtypeStruct((G, BT, K), jnp.float32),
        jax.ShapeDtypeStruct((G, BT, K), jnp.float32),
        jax.ShapeDtypeStruct((G * IR, 128), jnp.int32),
        jax.ShapeDtypeStruct((1, 1), jnp.float32),
    ],
)


def _sc_body(idxoff_hbm, cv_hbm, quant_hbm, idxoff_v, rows_v, sem):
    wid = lax.axis_index("c") * 16 + lax.axis_index("s")
    base = wid * PPW
    g = wid // 16
    b = (wid % 16) // 8
    t0 = (wid % 8) * PPW
    pltpu.sync_copy(idxoff_hbm.at[pl.ds(base, PPW)], idxoff_v)
    pltpu.async_copy(cv_hbm.at[idxoff_v], rows_v, sem).wait()
    pltpu.sync_copy(rows_v,
                    quant_hbm.at[b, pl.ds(t0, PPW), pl.ds(g * GD, GD)])


_sc_call = functools.partial(
    pl.kernel,
    mesh=plsc.VectorSubcoreMesh(core_axis_name="c", subcore_axis_name="s"),
    out_type=jax.ShapeDtypeStruct((B, T, D), jnp.float32),
    scratch_types=[
        pltpu.VMEM((PPW,), jnp.int32),
        pltpu.VMEM((PPW, GD), jnp.float32),
        pltpu.SemaphoreType.DMA,
    ],
)(_sc_body)


def kernel(hidden_states, W, b, codevectors):
    hs2 = hidden_states.reshape(BT, DIN)
    cv2 = codevectors.reshape(G * K, GD)
    dist, enc, idxoff, ppl = _tc_call(hs2, W, b, codevectors)
    quantized = _sc_call(idxoff.reshape(G * BT), cv2)
    distances = dist.reshape(G, B, T, K)
    encodings = enc.reshape(G, B, T, K)
    return (quantized, encodings, distances, ppl[0, 0])
